# Initial kernel scaffold; baseline (speedup 1.0000x reference)
import jax
import jax.numpy as jnp
from jax.experimental import pallas as pl


def kernel(x, emb, W_ih, W_hh, b_ih, b_hh, W_l, b_l):
    raise NotImplementedError("write your pallas kernel here")


# speedup vs baseline: 7.3116x; 7.3116x over previous
import functools

import jax
import jax.numpy as jnp
from jax import lax
from jax.experimental import pallas as pl
from jax.experimental.pallas import tpu as pltpu
from jax.experimental.pallas import tpu_sc as plsc

_V = 100000
_E = 200
_H = 32
_G = 4 * _H
_L = 50
_N = 4096

_ROW_BLK = 1000

_NC, _NS = 2, 16
_NW = _NC * _NS
_TOT = _L * _N
_RPW = _TOT // _NW
_CH = _RPW // 128


def _proj_body(emb_ref, wt_ref, b_ref, out_ref):
    out_ref[...] = (
        jnp.dot(emb_ref[...], wt_ref[...], preferred_element_type=jnp.float32)
        + b_ref[...]
    )


def _project_table(emb, W_ih, b_ih, b_hh):
    wt = W_ih.T
    b = (b_ih + b_hh).reshape(1, _G)
    return pl.pallas_call(
        _proj_body,
        grid=(_V // _ROW_BLK,),
        in_specs=[
            pl.BlockSpec((_ROW_BLK, _E), lambda i: (i, 0)),
            pl.BlockSpec((_E, _G), lambda i: (0, 0)),
            pl.BlockSpec((1, _G), lambda i: (0, 0)),
        ],
        out_specs=pl.BlockSpec((_ROW_BLK, _G), lambda i: (i, 0)),
        out_shape=jax.ShapeDtypeStruct((_V, _G), jnp.float32),
    )(emb, wt, b)


def _gather_body(p_hbm, x_hbm, out_hbm, idx_v, rows_v, sem):
    wid = lax.axis_index("s") * _NC + lax.axis_index("c")
    pltpu.sync_copy(x_hbm.at[wid], idx_v)
    base = wid * _RPW

    def body(r, carry):
        pltpu.async_copy(p_hbm.at[idx_v.at[r]], rows_v, sem).wait()
        pltpu.sync_copy(rows_v, out_hbm.at[pl.ds(base + r * 128, 128)])
        return carry

    lax.fori_loop(0, _CH, body, 0)


def _gather(p, x_flat):
    x3 = x_flat.reshape(_NW, _CH, 128)
    mesh = plsc.VectorSubcoreMesh(core_axis_name="c", subcore_axis_name="s")
    fn = functools.partial(
        pl.kernel,
        mesh=mesh,
        out_type=jax.ShapeDtypeStruct((_TOT, _G), jnp.float32),
        scratch_types=[
            pltpu.VMEM((_CH, 128), jnp.int32),
            pltpu.VMEM((128, _G), jnp.float32),
            pltpu.SemaphoreType.DMA,
        ],
    )(_gather_body)
    return fn(p, x3)


def _lstm_body(g_ref, whh_ref, wl_ref, bl_ref, out_ref, h_ref, c_ref):
    t = pl.program_id(0)

    @pl.when(t == 0)
    def _():
        h_ref[...] = jnp.zeros_like(h_ref)
        c_ref[...] = jnp.zeros_like(c_ref)

    h = h_ref[...]
    gates = g_ref[0] + jnp.dot(h, whh_ref[...], preferred_element_type=jnp.float32)
    i = jax.nn.sigmoid(gates[:, 0:_H])
    f = jax.nn.sigmoid(gates[:, _H:2 * _H])
    g = jnp.tanh(gates[:, 2 * _H:3 * _H])
    o = jax.nn.sigmoid(gates[:, 3 * _H:4 * _H])
    c = f * c_ref[...] + i * g
    h = o * jnp.tanh(c)
    c_ref[...] = c
    h_ref[...] = h
    logits = jnp.dot(h, wl_ref[...], preferred_element_type=jnp.float32) + bl_ref[...]
    m = jnp.max(logits, axis=0, keepdims=True)
    lse = m + jnp.log(jnp.sum(jnp.exp(logits - m), axis=0, keepdims=True))
    out_ref[0] = logits - lse


def _lstm(g, W_hh, W_l, b_l):
    whh_t = W_hh.T
    wl_t = W_l.T
    bl = b_l.reshape(1, 2)
    return pl.pallas_call(
        _lstm_body,
        grid=(_L,),
        in_specs=[
            pl.BlockSpec((1, _N, _G), lambda t: (t, 0, 0)),
            pl.BlockSpec((_H, _G), lambda t: (0, 0)),
            pl.BlockSpec((_H, 2), lambda t: (0, 0)),
            pl.BlockSpec((1, 2), lambda t: (0, 0)),
        ],
        out_specs=pl.BlockSpec((1, _N, 2), lambda t: (t, 0, 0)),
        out_shape=jax.ShapeDtypeStruct((_L, _N, 2), jnp.float32),
        scratch_shapes=[
            pltpu.VMEM((_N, _H), jnp.float32),
            pltpu.VMEM((_N, _H), jnp.float32),
        ],
    )(g, whh_t, wl_t, bl)


def kernel(x, emb, W_ih, W_hh, b_ih, b_hh, W_l, b_l):
    p = _project_table(emb, W_ih, b_ih, b_hh)
    g = _gather(p, x.reshape(-1))
    g = g.reshape(_L, _N, _G)
    return _lstm(g, W_hh, W_l, b_l)
